# baseline (speedup 1.0000x reference)
import dataclasses
import functools

import jax
import jax.numpy as jnp
from jax import lax
from jax.experimental import pallas as pl
from jax.experimental.pallas import tpu as pltpu
from jax.experimental.pallas import tpu_sc as plsc

N = 10000
D = 128
E = 320000
NC = 2
NS = 16
NW = NC * NS
CPT = D // NW
CPT_A = D // NS
EH = E // NC
EP = E // NW
SCH_A = 4000
SCH_B = 4000

_mesh = plsc.VectorSubcoreMesh(core_axis_name="c", subcore_axis_name="s")

_sc_params = pltpu.CompilerParams()
if "needs_layout_passes" in pltpu.CompilerParams.__dataclass_fields__:
    _sc_params = dataclasses.replace(_sc_params, needs_layout_passes=False)


def _normalize_body(x_ref, y_ref):
    xb = x_ref[...]
    n2 = jnp.sum(xb * xb, axis=1, keepdims=True)
    inv = jnp.where(n2 > 0, lax.rsqrt(n2), 0.0)
    y_ref[...] = xb * inv


def _tc_normalize(x):
    return pl.pallas_call(
        _normalize_body,
        out_shape=jax.ShapeDtypeStruct((N, D), jnp.float32),
        grid=(10,),
        in_specs=[pl.BlockSpec((N // 10, D), lambda i: (i, 0))],
        out_specs=pl.BlockSpec((N // 10, D), lambda i: (i, 0)),
    )(x)


@functools.partial(
    pl.kernel,
    mesh=_mesh,
    compiler_params=_sc_params,
    out_type=jax.ShapeDtypeStruct((NS * E,), jnp.float32),
    scratch_types=[
        pltpu.VMEM((CPT_A, N), jnp.float32),
        pltpu.VMEM((SCH_A,), jnp.int32),
        pltpu.VMEM((SCH_A,), jnp.int32),
        pltpu.VMEM((SCH_A,), jnp.int32),
        pltpu.VMEM((SCH_A,), jnp.int32),
        pltpu.VMEM((SCH_A,), jnp.float32),
        pltpu.VMEM((SCH_A,), jnp.float32),
        pltpu.SemaphoreType.DMA,
        pltpu.SemaphoreType.DMA,
        pltpu.SemaphoreType.DMA,
        pltpu.SemaphoreType.DMA,
        pltpu.SemaphoreType.DMA,
        pltpu.SemaphoreType.DMA,
    ],
)
def _sc_dots(yt_hbm, row_hbm, col_hbm, part_hbm, ytloc,
             rowb0, rowb1, colb0, colb1, pbuf0, pbuf1,
             sr0, sr1, sc0, sc1, sp0, sp1):
    cid = lax.axis_index("c")
    sid = lax.axis_index("s")
    ebase = cid * EH
    nsch = EH // SCH_A

    pltpu.sync_copy(yt_hbm.at[pl.ds(sid * CPT_A, CPT_A)], ytloc)

    def _in_copies(sc, rb, cb, sr, scm):
        off = ebase + sc * SCH_A
        return (pltpu.make_async_copy(row_hbm.at[pl.ds(off, SCH_A)], rb, sr),
                pltpu.make_async_copy(col_hbm.at[pl.ds(off, SCH_A)], cb, scm))

    def _out_copy(sc, pb, sp):
        off = sc * SCH_A
        return pltpu.make_async_copy(
            pb, part_hbm.at[pl.ds(sid * E + ebase + off, SCH_A)], sp)

    def _start_in(sc, rb, cb, sr, scm):
        off = ebase + sc * SCH_A
        pltpu.async_copy(row_hbm.at[pl.ds(off, SCH_A)], rb, sr)
        pltpu.async_copy(col_hbm.at[pl.ds(off, SCH_A)], cb, scm)

    def _compute(rb, cb, pb):
        @plsc.parallel_loop(0, SCH_A // 16, unroll=8)
        def _group(g):
            e16 = g * 16
            ridx = rb[pl.ds(e16, 16)]
            cidx = cb[pl.ds(e16, 16)]
            acc = jnp.zeros((16,), jnp.float32)
            for j in range(CPT_A):
                jv = jnp.full((16,), j, jnp.int32)
                va = plsc.load_gather(ytloc, [jv, ridx])
                vb = plsc.load_gather(ytloc, [jv, cidx])
                acc = acc + va * vb
            pb[pl.ds(e16, 16)] = acc

    _start_in(0, rowb0, colb0, sr0, sc0)

    @pl.loop(0, nsch)
    def _sch(sc):
        @pl.when(sc % 2 == 0)
        def _even():
            @pl.when(sc + 1 < nsch)
            def _pf():
                _start_in(sc + 1, rowb1, colb1, sr1, sc1)
            for c in _in_copies(sc, rowb0, colb0, sr0, sc0):
                c.wait()

            @pl.when(sc >= 2)
            def _wo():
                _out_copy(sc - 2, pbuf0, sp0).wait()
            _compute(rowb0, colb0, pbuf0)
            pltpu.async_copy(
                pbuf0, part_hbm.at[pl.ds(sid * E + ebase + sc * SCH_A, SCH_A)],
                sp0)

        @pl.when(sc % 2 == 1)
        def _odd():
            @pl.when(sc + 1 < nsch)
            def _pf():
                _start_in(sc + 1, rowb0, colb0, sr0, sc0)
            for c in _in_copies(sc, rowb1, colb1, sr1, sc1):
                c.wait()

            @pl.when(sc >= 2)
            def _wo():
                _out_copy(sc - 2, pbuf1, sp1).wait()
            _compute(rowb1, colb1, pbuf1)
            pltpu.async_copy(
                pbuf1, part_hbm.at[pl.ds(sid * E + ebase + sc * SCH_A, SCH_A)],
                sp1)

    _out_copy(nsch - 2, pbuf0, sp0).wait()
    _out_copy(nsch - 1, pbuf1, sp1).wait()


def _exp_body(b_ref, p_ref, ex_ref):
    s = jnp.sum(p_ref[...], axis=0, keepdims=True)
    ex_ref[...] = jnp.exp(b_ref[0, 0] * s)


def _tc_exp(part, beta2d):
    nblk = 20
    return pl.pallas_call(
        _exp_body,
        out_shape=jax.ShapeDtypeStruct((1, E), jnp.float32),
        grid=(nblk,),
        in_specs=[
            pl.BlockSpec((1, 1), lambda i: (0, 0)),
            pl.BlockSpec((NS, E // nblk), lambda i: (0, i)),
        ],
        out_specs=pl.BlockSpec((1, E // nblk), lambda i: (0, i)),
    )(beta2d, part)


@functools.partial(
    pl.kernel,
    mesh=_mesh,
    compiler_params=_sc_params,
    out_type=jax.ShapeDtypeStruct((NW, N), jnp.float32),
    scratch_types=[
        pltpu.VMEM((EP,), jnp.int32),
        pltpu.VMEM((EP,), jnp.float32),
        pltpu.VMEM((N,), jnp.float32),
    ],
)
def _sc_denom(row_hbm, ex_hbm, dpart_hbm, rowb, exb, dloc):
    cid = lax.axis_index("c")
    sid = lax.axis_index("s")
    wid = cid * NS + sid
    base = wid * EP

    pltpu.sync_copy(row_hbm.at[pl.ds(base, EP)], rowb)
    pltpu.sync_copy(ex_hbm.at[pl.ds(base, EP)], exb)

    @pl.loop(0, N, step=16)
    def _zero(i):
        dloc[pl.ds(i, 16)] = jnp.zeros((16,), jnp.float32)

    @plsc.parallel_loop(0, EP // 16, unroll=8)
    def _group(g):
        e16 = g * 16
        ridx = rowb[pl.ds(e16, 16)]
        ex = exb[pl.ds(e16, 16)]
        plsc.addupdate_scatter(dloc, [ridx], ex)

    pltpu.sync_copy(dloc, dpart_hbm.at[wid])


def _invdenom_body(dp_ref, inv_ref):
    s = jnp.sum(dp_ref[...], axis=0, keepdims=True)
    inv_ref[...] = 1.0 / s


def _tc_invdenom(dpart):
    return pl.pallas_call(
        _invdenom_body,
        out_shape=jax.ShapeDtypeStruct((1, N), jnp.float32),
        grid=(1,),
        in_specs=[pl.BlockSpec((NW, N), lambda i: (0, 0))],
        out_specs=pl.BlockSpec((1, N), lambda i: (0, 0)),
    )(dpart)


@functools.partial(
    pl.kernel,
    mesh=_mesh,
    compiler_params=_sc_params,
    out_type=jax.ShapeDtypeStruct((D, N), jnp.float32),
    scratch_types=[
        pltpu.VMEM((CPT, N), jnp.float32),
        pltpu.VMEM((CPT, N), jnp.float32),
        pltpu.VMEM((N,), jnp.float32),
        pltpu.VMEM((SCH_B,), jnp.int32),
        pltpu.VMEM((SCH_B,), jnp.int32),
        pltpu.VMEM((SCH_B,), jnp.int32),
        pltpu.VMEM((SCH_B,), jnp.int32),
        pltpu.VMEM((SCH_B,), jnp.float32),
        pltpu.VMEM((SCH_B,), jnp.float32),
        pltpu.SemaphoreType.DMA,
        pltpu.SemaphoreType.DMA,
        pltpu.SemaphoreType.DMA,
        pltpu.SemaphoreType.DMA,
        pltpu.SemaphoreType.DMA,
        pltpu.SemaphoreType.DMA,
    ],
)
def _sc_scatter(xt_hbm, row_hbm, col_hbm, ex_hbm, invd_hbm, outt_hbm,
                xtloc, otloc, invloc, rowb0, rowb1, colb0, colb1, exb0, exb1,
                sr0, sr1, sc0, sc1, se0, se1):
    cid = lax.axis_index("c")
    sid = lax.axis_index("s")
    wid = cid * NS + sid
    nsch = E // SCH_B

    pltpu.sync_copy(xt_hbm.at[pl.ds(wid * CPT, CPT)], xtloc)
    pltpu.sync_copy(invd_hbm.at[0], invloc)

    @pl.loop(0, N, step=16)
    def _zero(i):
        for j in range(CPT):
            otloc[j, pl.ds(i, 16)] = jnp.zeros((16,), jnp.float32)

    def _in_copies(sc, rb, cb, eb, sr, scm, se):
        off = sc * SCH_B
        return (pltpu.make_async_copy(row_hbm.at[pl.ds(off, SCH_B)], rb, sr),
                pltpu.make_async_copy(col_hbm.at[pl.ds(off, SCH_B)], cb, scm),
                pltpu.make_async_copy(ex_hbm.at[pl.ds(off, SCH_B)], eb, se))

    def _start_in(sc, rb, cb, eb, sr, scm, se):
        off = sc * SCH_B
        pltpu.async_copy(row_hbm.at[pl.ds(off, SCH_B)], rb, sr)
        pltpu.async_copy(col_hbm.at[pl.ds(off, SCH_B)], cb, scm)
        pltpu.async_copy(ex_hbm.at[pl.ds(off, SCH_B)], eb, se)

    def _compute(rb, cb, eb):
        @plsc.parallel_loop(0, SCH_B // 16, unroll=8)
        def _group(g):
            e16 = g * 16
            ridx = rb[pl.ds(e16, 16)]
            cidx = cb[pl.ds(e16, 16)]
            w = eb[pl.ds(e16, 16)] * plsc.load_gather(invloc, [ridx])
            for j in range(CPT):
                jv = jnp.full((16,), j, jnp.int32)
                v = plsc.load_gather(xtloc, [jv, cidx]) * w
                plsc.addupdate_scatter(otloc, [jv, ridx], v)

    _start_in(0, rowb0, colb0, exb0, sr0, sc0, se0)

    @pl.loop(0, nsch)
    def _sch(sc):
        @pl.when(sc % 2 == 0)
        def _even():
            @pl.when(sc + 1 < nsch)
            def _pf():
                _start_in(sc + 1, rowb1, colb1, exb1, sr1, sc1, se1)
            for c in _in_copies(sc, rowb0, colb0, exb0, sr0, sc0, se0):
                c.wait()
            _compute(rowb0, colb0, exb0)

        @pl.when(sc % 2 == 1)
        def _odd():
            @pl.when(sc + 1 < nsch)
            def _pf():
                _start_in(sc + 1, rowb0, colb0, exb0, sr0, sc0, se0)
            for c in _in_copies(sc, rowb1, colb1, exb1, sr1, sc1, se1):
                c.wait()
            _compute(rowb1, colb1, exb1)

    pltpu.sync_copy(otloc, outt_hbm.at[pl.ds(wid * CPT, CPT)])


def kernel(x, beta, edge_index):
    row = edge_index[0].astype(jnp.int32)
    col = edge_index[1].astype(jnp.int32)
    beta2d = beta.reshape(1, 1).astype(jnp.float32)
    y = _tc_normalize(x)
    yt = y.T
    xt = x.T
    part = _sc_dots(yt, row, col)
    ex2d = _tc_exp(part.reshape(NS, E), beta2d)
    ex = ex2d.reshape(E)
    dpart = _sc_denom(row, ex)
    invd = _tc_invdenom(dpart)
    outt = _sc_scatter(xt, row, col, ex, invd)
    return outt.T
